# baseline (speedup 1.0000x reference)
import functools

import jax
import jax.numpy as jnp
from jax import lax
from jax.experimental import pallas as pl
from jax.experimental.pallas import tpu as pltpu
from jax.experimental.pallas import tpu_sc as plsc

VOCAB = 1000000
SEQ = 200
D = 64
BATCH = 4096
NW = 32
BPW = BATCH // NW
NBUF = 4
LOOK = 2


def _impl_body(x_hbm, tok_hbm, pos_hbm, out_hbm, idx_v, pos_v, gbuf, tbuf,
               *sems):
    gsem = sems[:NBUF]
    wsem = sems[NBUF:]
    wid = lax.axis_index("s") * 2 + lax.axis_index("c")

    pltpu.sync_copy(x_hbm.at[:, pl.ds(wid * BPW, BPW)], idx_v)
    pltpu.sync_copy(pos_hbm, pos_v)

    iota = lax.broadcasted_iota(jnp.int32, (16,), 0)
    rows = [iota + g * 16 for g in range(BPW // 16)]

    def issue_gather(c, b):
        pltpu.async_copy(tok_hbm.at[idx_v.at[c]], gbuf.at[b], gsem[b])

    def wait_gather(c, b):
        pltpu.make_async_copy(tok_hbm.at[idx_v.at[c]], gbuf.at[b],
                              gsem[b]).wait()

    def issue_write(c, b):
        for dt in range(D // 8):
            pltpu.async_copy(tbuf.at[b, pl.ds(dt * 8, 8), :],
                             out_hbm.at[c, dt, wid], wsem[b])

    def wait_write(c, b):
        for dt in range(D // 8):
            pltpu.make_async_copy(tbuf.at[b, pl.ds(dt * 8, 8), :],
                                  out_hbm.at[c, dt, wid], wsem[b]).wait()

    def transpose_add(c, b):
        sv = jnp.full((16,), c, jnp.int32)

        def drow(d, carry):
            dv = jnp.full((16,), d, jnp.int32)
            psplat = plsc.load_gather(pos_v, [sv, dv])
            for g in range(BPW // 16):
                vals = plsc.load_gather(gbuf.at[b], [rows[g], dv])
                tbuf[b, d, pl.ds(g * 16, 16)] = vals + psplat
            return carry

        lax.fori_loop(0, D, drow, 0, unroll=4)

    for b in range(LOOK):
        issue_gather(b, b)

    def trip(t, carry):
        c0 = t * NBUF
        for b in range(NBUF):
            c = c0 + b
            nb = (b + LOOK) % NBUF
            cn = c + LOOK

            @pl.when(cn < SEQ)
            def _():
                @pl.when(c >= LOOK)
                def _():
                    wait_write(c - LOOK, nb)
                issue_gather(cn, nb)

            wait_gather(c, b)
            transpose_add(c, b)
            issue_write(c, b)
        return carry

    lax.fori_loop(0, SEQ // NBUF, trip, 0)

    for c in range(SEQ - LOOK, SEQ):
        wait_write(c, c % NBUF)


_impl = functools.partial(
    pl.kernel,
    out_type=jax.ShapeDtypeStruct((SEQ, D // 8, NW, 8, BPW), jnp.float32),
    mesh=plsc.VectorSubcoreMesh(core_axis_name="c", subcore_axis_name="s"),
    compiler_params=pltpu.CompilerParams(use_tc_tiling_on_sc=False,
                                         needs_layout_passes=False),
    scratch_types=[
        pltpu.VMEM((SEQ, BPW), jnp.int32),
        pltpu.VMEM((SEQ, D), jnp.float32),
        pltpu.VMEM((NBUF, BPW, D), jnp.float32),
        pltpu.VMEM((NBUF, D, BPW), jnp.float32),
    ] + [pltpu.SemaphoreType.DMA] * (2 * NBUF),
)(_impl_body)


def kernel(x, token_table, pos_table):
    tokp = jnp.pad(token_table, ((0, 0), (0, D))).reshape(2 * VOCAB, D)
    out5 = _impl((x * 2).T, tokp, pos_table)
    return out5.transpose(2, 4, 0, 1, 3).reshape(BATCH, SEQ, D)


# speedup vs baseline: 1.8228x; 1.8228x over previous
import functools

import jax
import jax.numpy as jnp
from jax import lax
from jax.experimental import pallas as pl
from jax.experimental.pallas import tpu as pltpu
from jax.experimental.pallas import tpu_sc as plsc

VOCAB = 1000000
SEQ = 200
D = 64
BATCH = 4096
NW = 32
BPW = BATCH // NW
NBUF = 4
LOOK = 2


def _impl_body(x_hbm, tok_hbm, pos_hbm, out_hbm, idx_v, pos_v, gbuf, tbuf,
               *sems):
    gsem = sems[:NBUF]
    wsem = sems[NBUF:]
    wid = lax.axis_index("s") * 2 + lax.axis_index("c")

    pltpu.sync_copy(x_hbm.at[:, pl.ds(wid * BPW, BPW)], idx_v)
    pltpu.sync_copy(pos_hbm, pos_v)

    iota = lax.broadcasted_iota(jnp.int32, (16,), 0)
    rowk = [iota + k * 16 for k in range(D // 16)]

    def issue_gather(c, b):
        pltpu.async_copy(tok_hbm.at[idx_v.at[c]], gbuf.at[b], gsem[b])

    def wait_gather(c, b):
        pltpu.make_async_copy(tok_hbm.at[idx_v.at[c]], gbuf.at[b],
                              gsem[b]).wait()

    def issue_write(c, b):
        for dt in range(D // 8):
            pltpu.async_copy(tbuf.at[b, pl.ds(dt * 8, 8), pl.ds(0, BPW)],
                             out_hbm.at[c, dt, wid], wsem[b])

    def wait_write(c, b):
        for dt in range(D // 8):
            pltpu.make_async_copy(tbuf.at[b, pl.ds(dt * 8, 8), pl.ds(0, BPW)],
                                  out_hbm.at[c, dt, wid], wsem[b]).wait()

    def transpose_add(c, b):
        posk = [pos_v[c, pl.ds(k * 16, 16)] for k in range(D // 16)]

        def row(r, carry):
            cv = jnp.full((16,), r, jnp.int32)
            for k in range(D // 16):
                vals = gbuf[b, r, pl.ds(k * 16, 16)] + posk[k]
                plsc.store_scatter(tbuf.at[b], [rowk[k], cv], vals)
            return carry

        lax.fori_loop(0, BPW, row, 0, unroll=4)

    for b in range(LOOK):
        issue_gather(b, b)

    def trip(t, carry):
        c0 = t * NBUF
        for b in range(NBUF):
            c = c0 + b
            nb = (b + LOOK) % NBUF
            cn = c + LOOK

            @pl.when(cn < SEQ)
            def _():
                @pl.when(c >= LOOK)
                def _():
                    wait_write(c - LOOK, nb)
                issue_gather(cn, nb)

            wait_gather(c, b)
            transpose_add(c, b)
            issue_write(c, b)
        return carry

    lax.fori_loop(0, SEQ // NBUF, trip, 0)

    for c in range(SEQ - LOOK, SEQ):
        wait_write(c, c % NBUF)


_impl = functools.partial(
    pl.kernel,
    out_type=jax.ShapeDtypeStruct((SEQ, D // 8, NW, 8, BPW), jnp.float32),
    mesh=plsc.VectorSubcoreMesh(core_axis_name="c", subcore_axis_name="s"),
    compiler_params=pltpu.CompilerParams(use_tc_tiling_on_sc=False,
                                         needs_layout_passes=False),
    scratch_types=[
        pltpu.VMEM((SEQ, BPW), jnp.int32),
        pltpu.VMEM((SEQ, D), jnp.float32),
        pltpu.VMEM((NBUF, BPW, D), jnp.float32),
        pltpu.VMEM((NBUF, D, BPW + 1), jnp.float32),
    ] + [pltpu.SemaphoreType.DMA] * (2 * NBUF),
)(_impl_body)


def kernel(x, token_table, pos_table):
    tokp = jnp.pad(token_table, ((0, 0), (0, D))).reshape(2 * VOCAB, D)
    out5 = _impl((x * 2).T, tokp, pos_table)
    return out5.transpose(2, 4, 0, 1, 3).reshape(BATCH, SEQ, D)


# speedup vs baseline: 1.8238x; 1.0005x over previous
import functools

import jax
import jax.numpy as jnp
from jax import lax
from jax.experimental import pallas as pl
from jax.experimental.pallas import tpu as pltpu
from jax.experimental.pallas import tpu_sc as plsc

VOCAB = 1000000
SEQ = 200
D = 64
BATCH = 4096
NW = 32
BPW = BATCH // NW
NBUF = 4
LOOK = 2


def _impl_body(x_hbm, tok_hbm, pos_hbm, out_hbm, idx_v, pos_v, gbuf, tbuf,
               *sems):
    gsem = sems[:NBUF]
    wsem = sems[NBUF:]
    wid = lax.axis_index("s") * 2 + lax.axis_index("c")

    pltpu.sync_copy(x_hbm.at[:, pl.ds(wid * BPW, BPW)], idx_v)
    pltpu.sync_copy(pos_hbm, pos_v)

    iota = lax.broadcasted_iota(jnp.int32, (16,), 0)
    rowk = [iota + k * 16 for k in range(D // 16)]

    def issue_gather(c, b):
        pltpu.async_copy(tok_hbm.at[idx_v.at[c]], gbuf.at[b], gsem[b])

    def wait_gather(c, b):
        pltpu.make_async_copy(tok_hbm.at[idx_v.at[c]], gbuf.at[b],
                              gsem[b]).wait()

    def issue_write(c, b):
        for dt in range(D // 8):
            pltpu.async_copy(tbuf.at[b, pl.ds(dt * 8, 8), pl.ds(0, BPW)],
                             out_hbm.at[c, dt, wid], wsem[b])

    def wait_write(c, b):
        for dt in range(D // 8):
            pltpu.make_async_copy(tbuf.at[b, pl.ds(dt * 8, 8), pl.ds(0, BPW)],
                                  out_hbm.at[c, dt, wid], wsem[b]).wait()

    def transpose_add(c, b):
        posk = [pos_v[c, pl.ds(k * 16, 16)] for k in range(D // 16)]

        def row(r, carry):
            cv = jnp.full((16,), r, jnp.int32)
            for k in range(D // 16):
                vals = gbuf[b, r, pl.ds(k * 16, 16)] + posk[k]
                plsc.store_scatter(tbuf.at[b], [rowk[k], cv], vals)
            return carry

        lax.fori_loop(0, BPW, row, 0, unroll=4)

    for b in range(LOOK):
        issue_gather(b, b)

    def trip(t, carry):
        c0 = t * NBUF
        for b in range(NBUF):
            c = c0 + b
            nb = (b + LOOK) % NBUF
            cn = c + LOOK

            @pl.when(cn < SEQ)
            def _():
                @pl.when(c >= LOOK)
                def _():
                    wait_write(c - LOOK, nb)
                issue_gather(cn, nb)

            wait_gather(c, b)
            transpose_add(c, b)
            issue_write(c, b)
        return carry

    lax.fori_loop(0, SEQ // NBUF, trip, 0)

    for c in range(SEQ - LOOK, SEQ):
        wait_write(c, c % NBUF)


_impl = functools.partial(
    pl.kernel,
    out_type=jax.ShapeDtypeStruct((SEQ, D // 8, NW, 8, BPW), jnp.float32),
    mesh=plsc.VectorSubcoreMesh(core_axis_name="c", subcore_axis_name="s"),
    compiler_params=pltpu.CompilerParams(use_tc_tiling_on_sc=False,
                                         needs_layout_passes=False),
    scratch_types=[
        pltpu.VMEM((SEQ, BPW), jnp.int32),
        pltpu.VMEM((SEQ, D), jnp.float32),
        pltpu.VMEM((NBUF, BPW, D), jnp.float32),
        pltpu.VMEM((NBUF, D, BPW + 1), jnp.float32),
    ] + [pltpu.SemaphoreType.DMA] * (2 * NBUF),
)(_impl_body)


def kernel(x, token_table, pos_table):
    tokp = jnp.concatenate(
        [token_table, jnp.zeros((VOCAB, D), jnp.float32)],
        axis=1).reshape(2 * VOCAB, D)
    out5 = _impl((x * 2).T, tokp, pos_table)
    return out5.transpose(2, 4, 0, 1, 3).reshape(BATCH, SEQ, D)
